# baseline (speedup 1.0000x reference)
import functools

import jax
import jax.numpy as jnp
from jax import lax
from jax.experimental import pallas as pl
from jax.experimental.pallas import tpu as pltpu
from jax.experimental.pallas import tpu_sc as plsc

N = 10000
E = 320000
F = 128
H = 128
R = 8

NC = 2
NS = 16
NW = NC * NS
CH = 128
NCH = 80
EPAD = NW * NCH * CH
NCH_A = 48
NCH_B = 2 * NCH - NCH_A
NCHX = max(NCH_A, NCH_B)

NPAD = N + 240
ROWS_PER_TILE = NPAD // NS
CNTP = R * N + 128
CNT_PER_TILE = CNTP // NS

BN = 2000
NB = N // BN

_mesh = plsc.VectorSubcoreMesh(core_axis_name="c", subcore_axis_name="s")


@functools.partial(
    pl.kernel,
    out_type=jax.ShapeDtypeStruct((NC * CNTP,), jnp.float32),
    mesh=_mesh,
    scratch_types=[
        pltpu.VMEM((NCHX, CH), jnp.int32),
        pltpu.VMEM((CH,), jnp.float32),
        pltpu.VMEM((CNT_PER_TILE,), jnp.float32),
        pltpu.VMEM_SHARED((CNTP,), jnp.float32),
    ],
)
def _sc_count(ckey_hbm, cnt_hbm, ckey_v, ones_v, zbuf, acc_sh):
    c = lax.axis_index("c")
    s = lax.axis_index("s")
    wid = c * NS + s

    def zfill(i, carry):
        zbuf[pl.ds(i * 16, 16)] = jnp.zeros((16,), jnp.float32)
        return carry

    lax.fori_loop(0, CNT_PER_TILE // 16, zfill, 0)
    pltpu.sync_copy(zbuf, acc_sh.at[pl.ds(s * CNT_PER_TILE, CNT_PER_TILE)])
    pltpu.sync_copy(ckey_hbm.at[wid], ckey_v)
    for i in range(CH // 16):
        ones_v[pl.ds(i * 16, 16)] = jnp.full((16,), 1.0, jnp.float32)
    plsc.subcore_barrier()

    def body(ch, carry):
        pltpu.sync_copy(ones_v, acc_sh.at[ckey_v.at[ch]], add=True)
        return carry

    lax.fori_loop(0, NCHX, body, 0)
    plsc.subcore_barrier()
    pltpu.sync_copy(acc_sh.at[pl.ds(s * CNT_PER_TILE, CNT_PER_TILE)], zbuf)
    pltpu.sync_copy(
        zbuf, cnt_hbm.at[pl.ds(c * CNTP + s * CNT_PER_TILE, CNT_PER_TILE)]
    )


@functools.partial(
    pl.kernel,
    out_type=jax.ShapeDtypeStruct((NW, NCHX, CH), jnp.float32),
    mesh=_mesh,
    scratch_types=[
        pltpu.VMEM((NCHX, CH), jnp.int32),
        pltpu.VMEM((CH,), jnp.float32),
        pltpu.VMEM((NCHX, CH), jnp.float32),
        pltpu.SemaphoreType.DMA,
    ],
)
def _sc_scales(cnt_hbm, ckey_hbm, s_hbm, ckey_v, g_v, s_v, sem):
    c = lax.axis_index("c")
    s = lax.axis_index("s")
    wid = c * NS + s
    pltpu.sync_copy(ckey_hbm.at[wid], ckey_v)

    def chunk(ch, carry):
        pltpu.async_copy(cnt_hbm.at[ckey_v.at[ch]], g_v, sem).wait()
        for g in range(CH // 16):
            s_v[ch, pl.ds(g * 16, 16)] = 1.0 / jnp.maximum(
                g_v[pl.ds(g * 16, 16)], 1.0
            )
        return carry

    lax.fori_loop(0, NCHX, chunk, 0)
    pltpu.sync_copy(s_v, s_hbm.at[wid])


@functools.partial(
    pl.kernel,
    out_type=jax.ShapeDtypeStruct((NC * NPAD, H), jnp.float32),
    mesh=_mesh,
    scratch_types=[
        pltpu.VMEM((2, 2, CH), jnp.int32),
        pltpu.VMEM((2, CH), jnp.float32),
        pltpu.VMEM((2, CH, H), jnp.float32),
        pltpu.VMEM_SHARED((NPAD, H), jnp.float32),
        pltpu.SemaphoreType.DMA,
        pltpu.SemaphoreType.DMA,
    ],
)
def _sc_aggregate(tbl_hbm, ep_hbm, s_hbm, acc_hbm, pk_v, pks_v, rows_v,
                  acc_sh, sem0, sem1):
    c = lax.axis_index("c")
    s = lax.axis_index("s")
    wid = c * NS + s
    nstage = CH
    n_stages = ROWS_PER_TILE // CH
    gsem = (sem0, sem1)

    def zfill(i, carry):
        for g in range(H // 16):
            rows_v[0, i, pl.ds(g * 16, 16)] = jnp.zeros((16,), jnp.float32)
        return carry

    lax.fori_loop(0, nstage, zfill, 0)
    for q in range(n_stages):
        pltpu.sync_copy(
            rows_v.at[0],
            acc_sh.at[pl.ds(s * ROWS_PER_TILE + q * nstage, nstage)],
        )
    plsc.subcore_barrier()

    def fire(ch, b):
        pltpu.sync_copy(ep_hbm.at[wid, ch], pk_v.at[b])
        pltpu.sync_copy(s_hbm.at[wid, ch], pks_v.at[b])
        pltpu.async_copy(tbl_hbm.at[pk_v.at[b, 0]], rows_v.at[b], gsem[b])

    def process(ch, b):
        pltpu.make_async_copy(
            tbl_hbm.at[pk_v.at[b, 0]], rows_v.at[b], gsem[b]
        ).wait()

        def group(gi, carry2):
            s16 = pks_v[b, pl.ds(gi * 16, 16)]
            for e in range(16):
                sb = lax.gather(
                    s16,
                    jnp.full((16, 1), e, jnp.int32),
                    lax.GatherDimensionNumbers(
                        offset_dims=(),
                        collapsed_slice_dims=(0,),
                        start_index_map=(0,),
                    ),
                    slice_sizes=(1,),
                    mode=lax.GatherScatterMode.PROMISE_IN_BOUNDS,
                )
                row = gi * 16 + e
                for g in range(H // 16):
                    rows_v[b, row, pl.ds(g * 16, 16)] = (
                        rows_v[b, row, pl.ds(g * 16, 16)] * sb
                    )
            return carry2

        lax.fori_loop(0, CH // 16, group, 0)
        pltpu.sync_copy(rows_v.at[b], acc_sh.at[pk_v.at[b, 1]], add=True)

    nch_me = jnp.where(c == 0, NCH_A, NCH_B)
    fire(0, 0)

    def pair(i, carry):
        fire(2 * i + 1, 1)
        process(2 * i, 0)

        @pl.when(i < nch_me // 2 - 1)
        def _():
            fire(2 * i + 2, 0)

        process(2 * i + 1, 1)
        return carry

    lax.fori_loop(0, nch_me // 2, pair, 0)
    plsc.subcore_barrier()
    for q in range(n_stages):
        pltpu.sync_copy(
            acc_sh.at[pl.ds(s * ROWS_PER_TILE + q * nstage, nstage)],
            rows_v.at[0],
        )
        pltpu.sync_copy(
            rows_v.at[0],
            acc_hbm.at[
                pl.ds(c * NPAD + s * ROWS_PER_TILE + q * nstage, nstage)
            ],
        )


def _tc_cnt_sum_body(c_ref, o_ref):
    o_ref[...] = c_ref[0] + c_ref[1]


def _tc_cnt_sum(cnt):
    cnt2 = cnt.reshape(NC, CNTP // H, H)
    out = pl.pallas_call(
        _tc_cnt_sum_body,
        out_shape=jax.ShapeDtypeStruct((CNTP // H, H), jnp.float32),
    )(cnt2)
    return out.reshape(CNTP)


def _tc_table_body(x_ref, w_ref, o_ref):
    for r in range(R + 1):
        o_ref[r] = jnp.dot(
            x_ref[...], w_ref[r], preferred_element_type=jnp.float32
        )


def _tc_table(x, wcat):
    return pl.pallas_call(
        _tc_table_body,
        grid=(NB,),
        in_specs=[
            pl.BlockSpec((BN, F), lambda j: (j, 0)),
            pl.BlockSpec((R + 1, F, H), lambda j: (0, 0, 0)),
        ],
        out_specs=pl.BlockSpec((R + 1, BN, H), lambda j: (0, j, 0)),
        out_shape=jax.ShapeDtypeStruct((R + 1, N, H), jnp.float32),
        compiler_params=pltpu.CompilerParams(
            dimension_semantics=("arbitrary",),
        ),
    )(x, wcat)


def _tc_combine_body(tbl_ref, acc_ref, b_ref, o_ref):
    o_ref[...] = jnp.maximum(
        tbl_ref[0] + acc_ref[0] + acc_ref[1] + b_ref[...], 0.0
    )


def _tc_combine(tbl, accs, b):
    return pl.pallas_call(
        _tc_combine_body,
        grid=(NB,),
        in_specs=[
            pl.BlockSpec((1, BN, H), lambda j: (R, j, 0)),
            pl.BlockSpec((NC, BN, H), lambda j: (0, j, 0)),
            pl.BlockSpec((1, H), lambda j: (0, 0)),
        ],
        out_specs=pl.BlockSpec((BN, H), lambda j: (j, 0)),
        out_shape=jax.ShapeDtypeStruct((N, H), jnp.float32),
        compiler_params=pltpu.CompilerParams(
            dimension_semantics=("arbitrary",),
        ),
    )(tbl, accs, b)


def _layer(x, wcat, b, epack, sv):
    tbl = _tc_table(x, wcat)
    accs = _sc_aggregate(tbl.reshape((R + 1) * N, H), epack, sv)
    accs = accs.reshape(NC, NPAD, H)[:, :N, :]
    return _tc_combine(tbl, accs, b.reshape(1, H))


def kernel(x, edge_index, edge_type, W1, root1, b1, W2, root2, b2, W3, root3, b3):
    src = edge_index[0]
    dst = edge_index[1]
    gidx = edge_type * N + src
    ckey = edge_type * N + dst

    pad = EPAD - E
    gidx = jnp.concatenate([gidx, jnp.zeros((pad,), jnp.int32)])
    ckey = jnp.concatenate([ckey, jnp.full((pad,), R * N, jnp.int32)])
    dstp = jnp.concatenate([dst, jnp.full((pad,), N, jnp.int32)])

    def split_ab(flat, fill):
        ea = NS * NCH_A * CH
        a = flat[:ea].reshape(NS, NCH_A, CH)
        bb = flat[ea:].reshape(NS, NCH_B, CH)
        a = jnp.pad(a, ((0, 0), (0, NCHX - NCH_A), (0, 0)),
                    constant_values=fill)
        bb = jnp.pad(bb, ((0, 0), (0, NCHX - NCH_B), (0, 0)),
                     constant_values=fill)
        return jnp.concatenate([a, bb], axis=0)

    gidx = split_ab(gidx, 0)
    ckey = split_ab(ckey, R * N)
    dstp = split_ab(dstp, N)
    epack = jnp.concatenate(
        [gidx.reshape(NW, NCHX, 1, CH), dstp.reshape(NW, NCHX, 1, CH)], axis=2
    )

    cnt = _tc_cnt_sum(_sc_count(ckey))
    sv = _sc_scales(cnt, ckey)

    w1 = jnp.concatenate([W1, root1[None]], axis=0)
    w2 = jnp.concatenate([W2, root2[None]], axis=0)
    w3 = jnp.concatenate([W3, root3[None]], axis=0)

    h = _layer(x, w1, b1, epack, sv)
    h = _layer(h, w2, b2, epack, sv)
    h = _layer(h, w3, b3, epack, sv)
    return h


# speedup vs baseline: 1.5261x; 1.5261x over previous
import functools

import jax
import jax.numpy as jnp
from jax import lax
from jax.experimental import pallas as pl
from jax.experimental.pallas import tpu as pltpu
from jax.experimental.pallas import tpu_sc as plsc

N = 10000
E = 320000
F = 128
H = 128
R = 8

NC = 2
NS = 16
NW = NC * NS
CH = 128
NCH = 80
EPAD = NW * NCH * CH
NCH_A = 80
NCH_B = 2 * NCH - NCH_A
NCHX = max(NCH_A, NCH_B)

NPAD = N + 240
ROWS_PER_TILE = NPAD // NS
CNTP = R * N + 128
CNT_PER_TILE = CNTP // NS

BN = 2000
NB = N // BN

_mesh = plsc.VectorSubcoreMesh(core_axis_name="c", subcore_axis_name="s")


@functools.partial(
    pl.kernel,
    out_type=jax.ShapeDtypeStruct((NC * CNTP,), jnp.float32),
    mesh=_mesh,
    scratch_types=[
        pltpu.VMEM((NCHX, CH), jnp.int32),
        pltpu.VMEM((CH,), jnp.float32),
        pltpu.VMEM((CNT_PER_TILE,), jnp.float32),
        pltpu.VMEM_SHARED((CNTP,), jnp.float32),
    ],
)
def _sc_count(ckey_hbm, cnt_hbm, ckey_v, ones_v, zbuf, acc_sh):
    c = lax.axis_index("c")
    s = lax.axis_index("s")
    wid = c * NS + s

    def zfill(i, carry):
        zbuf[pl.ds(i * 16, 16)] = jnp.zeros((16,), jnp.float32)
        return carry

    lax.fori_loop(0, CNT_PER_TILE // 16, zfill, 0)
    pltpu.sync_copy(zbuf, acc_sh.at[pl.ds(s * CNT_PER_TILE, CNT_PER_TILE)])
    pltpu.sync_copy(ckey_hbm.at[wid], ckey_v)
    for i in range(CH // 16):
        ones_v[pl.ds(i * 16, 16)] = jnp.full((16,), 1.0, jnp.float32)
    plsc.subcore_barrier()

    def body(ch, carry):
        pltpu.sync_copy(ones_v, acc_sh.at[ckey_v.at[ch]], add=True)
        return carry

    lax.fori_loop(0, NCHX, body, 0)
    plsc.subcore_barrier()
    pltpu.sync_copy(acc_sh.at[pl.ds(s * CNT_PER_TILE, CNT_PER_TILE)], zbuf)
    pltpu.sync_copy(
        zbuf, cnt_hbm.at[pl.ds(c * CNTP + s * CNT_PER_TILE, CNT_PER_TILE)]
    )


@functools.partial(
    pl.kernel,
    out_type=jax.ShapeDtypeStruct((NW, NCHX, CH), jnp.float32),
    mesh=_mesh,
    scratch_types=[
        pltpu.VMEM((NCHX, CH), jnp.int32),
        pltpu.VMEM((CH,), jnp.float32),
        pltpu.VMEM((NCHX, CH), jnp.float32),
        pltpu.SemaphoreType.DMA,
    ],
)
def _sc_scales(cnt_hbm, ckey_hbm, s_hbm, ckey_v, g_v, s_v, sem):
    c = lax.axis_index("c")
    s = lax.axis_index("s")
    wid = c * NS + s
    pltpu.sync_copy(ckey_hbm.at[wid], ckey_v)

    def chunk(ch, carry):
        pltpu.async_copy(cnt_hbm.at[ckey_v.at[ch]], g_v, sem).wait()
        for g in range(CH // 16):
            s_v[ch, pl.ds(g * 16, 16)] = 1.0 / jnp.maximum(
                g_v[pl.ds(g * 16, 16)], 1.0
            )
        return carry

    lax.fori_loop(0, NCHX, chunk, 0)
    pltpu.sync_copy(s_v, s_hbm.at[wid])


@functools.partial(
    pl.kernel,
    out_type=jax.ShapeDtypeStruct((NC * NPAD, H), jnp.float32),
    mesh=_mesh,
    scratch_types=[
        pltpu.VMEM((2, 2, CH), jnp.int32),
        pltpu.VMEM((2, CH), jnp.float32),
        pltpu.VMEM((2, CH, H), jnp.float32),
        pltpu.VMEM_SHARED((NPAD, H), jnp.float32),
        pltpu.SemaphoreType.DMA,
        pltpu.SemaphoreType.DMA,
    ],
)
def _sc_aggregate(tbl_hbm, ep_hbm, s_hbm, acc_hbm, pk_v, pks_v, rows_v,
                  acc_sh, sem0, sem1):
    c = lax.axis_index("c")
    s = lax.axis_index("s")
    wid = c * NS + s
    nstage = CH
    n_stages = ROWS_PER_TILE // CH
    gsem = (sem0, sem1)

    def zfill(i, carry):
        for g in range(H // 16):
            rows_v[0, i, pl.ds(g * 16, 16)] = jnp.zeros((16,), jnp.float32)
        return carry

    lax.fori_loop(0, nstage, zfill, 0)
    for q in range(n_stages):
        pltpu.sync_copy(
            rows_v.at[0],
            acc_sh.at[pl.ds(s * ROWS_PER_TILE + q * nstage, nstage)],
        )
    plsc.subcore_barrier()

    def fire(ch, b):
        pltpu.sync_copy(ep_hbm.at[wid, ch], pk_v.at[b])
        pltpu.sync_copy(s_hbm.at[wid, ch], pks_v.at[b])
        pltpu.async_copy(tbl_hbm.at[pk_v.at[b, 0]], rows_v.at[b], gsem[b])

    def process(ch, b):
        pltpu.make_async_copy(
            tbl_hbm.at[pk_v.at[b, 0]], rows_v.at[b], gsem[b]
        ).wait()

        def group(gi, carry2):
            s16 = pks_v[b, pl.ds(gi * 16, 16)]
            for e in range(16):
                sb = lax.gather(
                    s16,
                    jnp.full((16, 1), e, jnp.int32),
                    lax.GatherDimensionNumbers(
                        offset_dims=(),
                        collapsed_slice_dims=(0,),
                        start_index_map=(0,),
                    ),
                    slice_sizes=(1,),
                    mode=lax.GatherScatterMode.PROMISE_IN_BOUNDS,
                )
                row = gi * 16 + e
                for g in range(H // 16):
                    rows_v[b, row, pl.ds(g * 16, 16)] = (
                        rows_v[b, row, pl.ds(g * 16, 16)] * sb
                    )
            return carry2

        lax.fori_loop(0, CH // 16, group, 0)
        pltpu.sync_copy(rows_v.at[b], acc_sh.at[pk_v.at[b, 1]], add=True)

    nch_me = jnp.where(c == 0, NCH_A, NCH_B)
    fire(0, 0)

    def pair(i, carry):
        fire(2 * i + 1, 1)
        process(2 * i, 0)

        @pl.when(i < nch_me // 2 - 1)
        def _():
            fire(2 * i + 2, 0)

        process(2 * i + 1, 1)
        return carry

    lax.fori_loop(0, nch_me // 2, pair, 0)
    plsc.subcore_barrier()
    for q in range(n_stages):
        pltpu.sync_copy(
            acc_sh.at[pl.ds(s * ROWS_PER_TILE + q * nstage, nstage)],
            rows_v.at[0],
        )
        pltpu.sync_copy(
            rows_v.at[0],
            acc_hbm.at[
                pl.ds(c * NPAD + s * ROWS_PER_TILE + q * nstage, nstage)
            ],
        )


def _tc_cnt_sum_body(c_ref, o_ref):
    o_ref[...] = c_ref[0] + c_ref[1]


def _tc_cnt_sum(cnt):
    cnt2 = cnt.reshape(NC, CNTP // H, H)
    out = pl.pallas_call(
        _tc_cnt_sum_body,
        out_shape=jax.ShapeDtypeStruct((CNTP // H, H), jnp.float32),
    )(cnt2)
    return out.reshape(CNTP)


def _tc_table_body(x_ref, w_ref, tb_ref, rt_ref):
    for r in range(R):
        tb_ref[r] = jnp.dot(
            x_ref[...], w_ref[r], preferred_element_type=jnp.float32
        )
    rt_ref[...] = jnp.dot(
        x_ref[...], w_ref[R], preferred_element_type=jnp.float32
    )


def _tc_table(x, wcat):
    return pl.pallas_call(
        _tc_table_body,
        grid=(NB,),
        in_specs=[
            pl.BlockSpec((BN, F), lambda j: (j, 0)),
            pl.BlockSpec((R + 1, F, H), lambda j: (0, 0, 0)),
        ],
        out_specs=[
            pl.BlockSpec((R, BN, H), lambda j: (0, j, 0)),
            pl.BlockSpec((BN, H), lambda j: (j, 0)),
        ],
        out_shape=[
            jax.ShapeDtypeStruct((R, N, H), jnp.float32),
            jax.ShapeDtypeStruct((N, H), jnp.float32),
        ],
        compiler_params=pltpu.CompilerParams(
            dimension_semantics=("arbitrary",),
        ),
    )(x, wcat)


def _tc_combine_body(rt_ref, acc_ref, b_ref, o_ref):
    o_ref[...] = jnp.maximum(
        rt_ref[...]
        + acc_ref[0].astype(jnp.float32)
        + acc_ref[1].astype(jnp.float32)
        + b_ref[...],
        0.0,
    )


def _tc_combine(rt, accs, b):
    return pl.pallas_call(
        _tc_combine_body,
        grid=(NB,),
        in_specs=[
            pl.BlockSpec((BN, H), lambda j: (j, 0)),
            pl.BlockSpec((NC, BN, H), lambda j: (0, j, 0)),
            pl.BlockSpec((1, H), lambda j: (0, 0)),
        ],
        out_specs=pl.BlockSpec((BN, H), lambda j: (j, 0)),
        out_shape=jax.ShapeDtypeStruct((N, H), jnp.float32),
        compiler_params=pltpu.CompilerParams(
            dimension_semantics=("arbitrary",),
        ),
    )(rt, accs, b)


def _layer(x, wcat, b, epack, sv):
    tbl, rt = _tc_table(x, wcat)
    accs = _sc_aggregate(tbl.reshape(R * N, H), epack, sv)
    accs = accs.reshape(NC, NPAD, H)[:, :N, :]
    return _tc_combine(rt, accs, b.reshape(1, H))


def kernel(x, edge_index, edge_type, W1, root1, b1, W2, root2, b2, W3, root3, b3):
    src = edge_index[0]
    dst = edge_index[1]
    gidx = edge_type * N + src
    ckey = edge_type * N + dst

    pad = EPAD - E
    gidx = jnp.concatenate([gidx, jnp.zeros((pad,), jnp.int32)])
    ckey = jnp.concatenate([ckey, jnp.full((pad,), R * N, jnp.int32)])
    dstp = jnp.concatenate([dst, jnp.full((pad,), N, jnp.int32)])

    def split_ab(flat, fill):
        ea = NS * NCH_A * CH
        a = flat[:ea].reshape(NS, NCH_A, CH)
        bb = flat[ea:].reshape(NS, NCH_B, CH)
        a = jnp.pad(a, ((0, 0), (0, NCHX - NCH_A), (0, 0)),
                    constant_values=fill)
        bb = jnp.pad(bb, ((0, 0), (0, NCHX - NCH_B), (0, 0)),
                     constant_values=fill)
        return jnp.concatenate([a, bb], axis=0)

    gidx = split_ab(gidx, 0)
    ckey = split_ab(ckey, R * N)
    dstp = split_ab(dstp, N)
    epack = jnp.concatenate(
        [gidx.reshape(NW, NCHX, 1, CH), dstp.reshape(NW, NCHX, 1, CH)], axis=2
    )

    cnt = _tc_cnt_sum(_sc_count(ckey))
    sv = _sc_scales(cnt, ckey)

    w1 = jnp.concatenate([W1, root1[None]], axis=0)
    w2 = jnp.concatenate([W2, root2[None]], axis=0)
    w3 = jnp.concatenate([W3, root3[None]], axis=0)

    h = _layer(x, w1, b1, epack, sv)
    h = _layer(h, w2, b2, epack, sv)
    h = _layer(h, w3, b3, epack, sv)
    return h


# speedup vs baseline: 1.5467x; 1.0135x over previous
import functools

import jax
import jax.numpy as jnp
from jax import lax
from jax.experimental import pallas as pl
from jax.experimental.pallas import tpu as pltpu
from jax.experimental.pallas import tpu_sc as plsc

N = 10000
E = 320000
F = 128
H = 128
R = 8

NC = 2
NS = 16
NW = NC * NS
CH = 128
NCH = 80
EPAD = NW * NCH * CH
NCH_A = 80
NCH_B = 2 * NCH - NCH_A
NCHX = max(NCH_A, NCH_B)

NPAD = N
RPT = 632
CNTP = R * N + 128
CNT_PER_TILE = CNTP // NS

BN = 2000
NB = N // BN

_mesh = plsc.VectorSubcoreMesh(core_axis_name="c", subcore_axis_name="s")


@functools.partial(
    pl.kernel,
    out_type=jax.ShapeDtypeStruct((NC * CNTP,), jnp.float32),
    mesh=_mesh,
    scratch_types=[
        pltpu.VMEM((NCHX, CH), jnp.int32),
        pltpu.VMEM((CH,), jnp.float32),
        pltpu.VMEM((CNT_PER_TILE,), jnp.float32),
        pltpu.VMEM_SHARED((CNTP,), jnp.float32),
    ],
)
def _sc_count(ckey_hbm, cnt_hbm, ckey_v, ones_v, zbuf, acc_sh):
    c = lax.axis_index("c")
    s = lax.axis_index("s")
    wid = c * NS + s

    def zfill(i, carry):
        zbuf[pl.ds(i * 16, 16)] = jnp.zeros((16,), jnp.float32)
        return carry

    lax.fori_loop(0, CNT_PER_TILE // 16, zfill, 0)
    pltpu.sync_copy(zbuf, acc_sh.at[pl.ds(s * CNT_PER_TILE, CNT_PER_TILE)])
    pltpu.sync_copy(ckey_hbm.at[wid], ckey_v)
    for i in range(CH // 16):
        ones_v[pl.ds(i * 16, 16)] = jnp.full((16,), 1.0, jnp.float32)
    plsc.subcore_barrier()

    def body(ch, carry):
        pltpu.sync_copy(ones_v, acc_sh.at[ckey_v.at[ch]], add=True)
        return carry

    lax.fori_loop(0, NCHX, body, 0)
    plsc.subcore_barrier()
    pltpu.sync_copy(acc_sh.at[pl.ds(s * CNT_PER_TILE, CNT_PER_TILE)], zbuf)
    pltpu.sync_copy(
        zbuf, cnt_hbm.at[pl.ds(c * CNTP + s * CNT_PER_TILE, CNT_PER_TILE)]
    )


@functools.partial(
    pl.kernel,
    out_type=jax.ShapeDtypeStruct((NW, NCHX, CH), jnp.float32),
    mesh=_mesh,
    scratch_types=[
        pltpu.VMEM((NCHX, CH), jnp.int32),
        pltpu.VMEM((CH,), jnp.float32),
        pltpu.VMEM((NCHX, CH), jnp.float32),
        pltpu.SemaphoreType.DMA,
    ],
)
def _sc_scales(cnt_hbm, ckey_hbm, s_hbm, ckey_v, g_v, s_v, sem):
    c = lax.axis_index("c")
    s = lax.axis_index("s")
    wid = c * NS + s
    pltpu.sync_copy(ckey_hbm.at[wid], ckey_v)

    def chunk(ch, carry):
        pltpu.async_copy(cnt_hbm.at[ckey_v.at[ch]], g_v, sem).wait()
        for g in range(CH // 16):
            cnt16 = g_v[pl.ds(g * 16, 16)]
            s_v[ch, pl.ds(g * 16, 16)] = jnp.where(
                cnt16 == 0.0, 0.0, 1.0 / jnp.maximum(cnt16, 1.0)
            )
        return carry

    lax.fori_loop(0, NCHX, chunk, 0)
    pltpu.sync_copy(s_v, s_hbm.at[wid])


@functools.partial(
    pl.kernel,
    out_type=jax.ShapeDtypeStruct((NC * NPAD, H), jnp.float32),
    mesh=_mesh,
    scratch_types=[
        pltpu.VMEM((3, 2, CH), jnp.int32),
        pltpu.VMEM((3, CH), jnp.float32),
        pltpu.VMEM((3, CH, H), jnp.float32),
        pltpu.VMEM_SHARED((NPAD, H), jnp.float32),
        pltpu.SemaphoreType.DMA,
        pltpu.SemaphoreType.DMA,
        pltpu.SemaphoreType.DMA,
    ],
)
def _sc_aggregate(tbl_hbm, ep_hbm, s_hbm, acc_hbm, pk_v, pks_v, rows_v,
                  acc_sh, sem0, sem1, sem2):
    c = lax.axis_index("c")
    s = lax.axis_index("s")
    wid = c * NS + s
    gsem = (sem0, sem1, sem2)

    def zfill(i, carry):
        for g in range(H // 16):
            rows_v[0, i, pl.ds(g * 16, 16)] = jnp.zeros((16,), jnp.float32)
        return carry

    lax.fori_loop(0, CH, zfill, 0)

    def stripe_copy(dump):
        for q in range(4):
            lo = s * RPT + q * CH
            if dump:
                pltpu.sync_copy(acc_sh.at[pl.ds(lo, CH)], rows_v.at[0])
                pltpu.sync_copy(rows_v.at[0], acc_hbm.at[pl.ds(c * NPAD + lo, CH)])
            else:
                pltpu.sync_copy(rows_v.at[0], acc_sh.at[pl.ds(lo, CH)])
        lo = s * RPT + 4 * CH

        @pl.when(s < NS - 1)
        def _():
            if dump:
                pltpu.sync_copy(acc_sh.at[pl.ds(lo, 120)], rows_v.at[0, :120])
                pltpu.sync_copy(
                    rows_v.at[0, :120], acc_hbm.at[pl.ds(c * NPAD + lo, 120)]
                )
            else:
                pltpu.sync_copy(rows_v.at[0, :120], acc_sh.at[pl.ds(lo, 120)])

        @pl.when(s == NS - 1)
        def _():
            if dump:
                pltpu.sync_copy(acc_sh.at[pl.ds(lo, 8)], rows_v.at[0, :8])
                pltpu.sync_copy(
                    rows_v.at[0, :8], acc_hbm.at[pl.ds(c * NPAD + lo, 8)]
                )
            else:
                pltpu.sync_copy(rows_v.at[0, :8], acc_sh.at[pl.ds(lo, 8)])

    stripe_copy(False)
    plsc.subcore_barrier()

    def fire(ch, b):
        pltpu.sync_copy(ep_hbm.at[wid, ch], pk_v.at[b])
        pltpu.sync_copy(s_hbm.at[wid, ch], pks_v.at[b])
        pltpu.async_copy(tbl_hbm.at[pk_v.at[b, 0]], rows_v.at[b], gsem[b])

    def process(ch, b):
        pltpu.make_async_copy(
            tbl_hbm.at[pk_v.at[b, 0]], rows_v.at[b], gsem[b]
        ).wait()

        def group(gi, carry2):
            s16 = pks_v[b, pl.ds(gi * 16, 16)]
            for e in range(16):
                sb = lax.gather(
                    s16,
                    jnp.full((16, 1), e, jnp.int32),
                    lax.GatherDimensionNumbers(
                        offset_dims=(),
                        collapsed_slice_dims=(0,),
                        start_index_map=(0,),
                    ),
                    slice_sizes=(1,),
                    mode=lax.GatherScatterMode.PROMISE_IN_BOUNDS,
                )
                row = gi * 16 + e
                for g in range(H // 16):
                    rows_v[b, row, pl.ds(g * 16, 16)] = (
                        rows_v[b, row, pl.ds(g * 16, 16)] * sb
                    )
            return carry2

        lax.fori_loop(0, CH // 16, group, 0)
        pltpu.sync_copy(rows_v.at[b], acc_sh.at[pk_v.at[b, 1]], add=True)

    nch_me = jnp.where(c == 0, NCH_A, NCH_B)
    fire(0, 0)
    fire(1, 1)

    def trip(i, carry):
        fire(3 * i + 2, 2)
        process(3 * i, 0)

        @pl.when(i < nch_me // 3 - 1)
        def _():
            fire(3 * i + 3, 0)

        process(3 * i + 1, 1)

        @pl.when(i < nch_me // 3 - 1)
        def _():
            fire(3 * i + 4, 1)

        process(3 * i + 2, 2)
        return carry

    lax.fori_loop(0, nch_me // 3, trip, 0)
    plsc.subcore_barrier()
    stripe_copy(True)


def _tc_cnt_sum_body(c_ref, o_ref):
    o_ref[...] = c_ref[0] + c_ref[1]


def _tc_cnt_sum(cnt):
    cnt2 = cnt.reshape(NC, CNTP // H, H)
    out = pl.pallas_call(
        _tc_cnt_sum_body,
        out_shape=jax.ShapeDtypeStruct((CNTP // H, H), jnp.float32),
    )(cnt2)
    return out.reshape(CNTP)


def _tc_table_body(x_ref, w_ref, tb_ref, rt_ref):
    for r in range(R):
        tb_ref[r] = jnp.dot(
            x_ref[...], w_ref[r], preferred_element_type=jnp.float32
        )
    rt_ref[...] = jnp.dot(
        x_ref[...], w_ref[R], preferred_element_type=jnp.float32
    )


def _tc_table(x, wcat):
    return pl.pallas_call(
        _tc_table_body,
        grid=(NB,),
        in_specs=[
            pl.BlockSpec((BN, F), lambda j: (j, 0)),
            pl.BlockSpec((R + 1, F, H), lambda j: (0, 0, 0)),
        ],
        out_specs=[
            pl.BlockSpec((R, BN, H), lambda j: (0, j, 0)),
            pl.BlockSpec((BN, H), lambda j: (j, 0)),
        ],
        out_shape=[
            jax.ShapeDtypeStruct((R, N, H), jnp.float32),
            jax.ShapeDtypeStruct((N, H), jnp.float32),
        ],
        compiler_params=pltpu.CompilerParams(
            dimension_semantics=("arbitrary",),
        ),
    )(x, wcat)


def _tc_combine_body(rt_ref, acc_ref, b_ref, o_ref):
    o_ref[...] = jnp.maximum(
        rt_ref[...]
        + acc_ref[0].astype(jnp.float32)
        + acc_ref[1].astype(jnp.float32)
        + b_ref[...],
        0.0,
    )


def _tc_combine(rt, accs, b):
    return pl.pallas_call(
        _tc_combine_body,
        grid=(NB,),
        in_specs=[
            pl.BlockSpec((BN, H), lambda j: (j, 0)),
            pl.BlockSpec((NC, BN, H), lambda j: (0, j, 0)),
            pl.BlockSpec((1, H), lambda j: (0, 0)),
        ],
        out_specs=pl.BlockSpec((BN, H), lambda j: (j, 0)),
        out_shape=jax.ShapeDtypeStruct((N, H), jnp.float32),
        compiler_params=pltpu.CompilerParams(
            dimension_semantics=("arbitrary",),
        ),
    )(rt, accs, b)


def _layer(x, wcat, b, epack, sv):
    tbl, rt = _tc_table(x, wcat)
    accs = _sc_aggregate(tbl.reshape(R * N, H), epack, sv)
    accs = accs.reshape(NC, NPAD, H)[:, :N, :]
    return _tc_combine(rt, accs, b.reshape(1, H))


def kernel(x, edge_index, edge_type, W1, root1, b1, W2, root2, b2, W3, root3, b3):
    src = edge_index[0]
    dst = edge_index[1]
    gidx = edge_type * N + src
    ckey = edge_type * N + dst

    pad = EPAD - E
    gidx = jnp.concatenate([gidx, jnp.zeros((pad,), jnp.int32)])
    ckey = jnp.concatenate([ckey, jnp.full((pad,), R * N, jnp.int32)])
    skey = jnp.concatenate([edge_type * N + dst, jnp.full((pad,), R * N + 64, jnp.int32)])
    dstp = jnp.concatenate([dst, jnp.zeros((pad,), jnp.int32)])

    def split_ab(flat, fill):
        ea = NS * NCH_A * CH
        a = flat[:ea].reshape(NS, NCH_A, CH)
        bb = flat[ea:].reshape(NS, NCH_B, CH)
        a = jnp.pad(a, ((0, 0), (0, NCHX - NCH_A), (0, 0)),
                    constant_values=fill)
        bb = jnp.pad(bb, ((0, 0), (0, NCHX - NCH_B), (0, 0)),
                     constant_values=fill)
        return jnp.concatenate([a, bb], axis=0)

    gidx = split_ab(gidx, 0)
    ckey = split_ab(ckey, R * N)
    skey = split_ab(skey, R * N + 64)
    dstp = split_ab(dstp, 0)
    epack = jnp.concatenate(
        [gidx.reshape(NW, NCHX, 1, CH), dstp.reshape(NW, NCHX, 1, CH)], axis=2
    )

    cnt = _tc_cnt_sum(_sc_count(ckey))
    sv = _sc_scales(cnt, skey)

    w1 = jnp.concatenate([W1, root1[None]], axis=0)
    w2 = jnp.concatenate([W2, root2[None]], axis=0)
    w3 = jnp.concatenate([W3, root3[None]], axis=0)

    h = _layer(x, w1, b1, epack, sv)
    h = _layer(h, w2, b2, epack, sv)
    h = _layer(h, w3, b3, epack, sv)
    return h
